# Initial kernel scaffold; baseline (speedup 1.0000x reference)
import functools

import jax
import jax.numpy as jnp
from jax import lax
from jax.experimental import pallas as pl
from jax.experimental.pallas import tpu as pltpu
from jax.experimental.pallas import tpu_sc as plsc

N = 10000
D = 128
E = 320000
NC = 2
NS = 16
NW = NC * NS
EW = E // NW
L = 16
K = 16
NB = EW // K
RPT = N // NS


def _dense_body(x_ref, wv_ref, wr_ref, bv_ref, aw_ref, ab_ref,
                v_ref, p_ref, q_ref):
    x = x_ref[...]
    v_ref[...] = lax.dot_general(
        x, wv_ref[...], (((1,), (1,)), ((), ())),
        precision=lax.Precision.HIGHEST,
        preferred_element_type=jnp.float32) + bv_ref[...][None, :]
    a1 = aw_ref[0:D]
    a2 = aw_ref[D:2 * D]
    w2 = jnp.dot(a2, wr_ref[...], precision=lax.Precision.HIGHEST,
                 preferred_element_type=jnp.float32)
    p_ref[...] = jnp.dot(x, a1, precision=lax.Precision.HIGHEST,
                         preferred_element_type=jnp.float32) + ab_ref[0]
    q_ref[...] = jnp.dot(x, w2, precision=lax.Precision.HIGHEST,
                         preferred_element_type=jnp.float32)


_dense = pl.pallas_call(
    _dense_body,
    out_shape=(jax.ShapeDtypeStruct((N, D), jnp.float32),
               jax.ShapeDtypeStruct((N,), jnp.float32),
               jax.ShapeDtypeStruct((N,), jnp.float32)),
    in_specs=[pl.BlockSpec(memory_space=pltpu.VMEM)] * 5
             + [pl.BlockSpec(memory_space=pltpu.SMEM)],
)


_mesh = plsc.VectorSubcoreMesh(core_axis_name="c", subcore_axis_name="s")


@functools.partial(
    pl.kernel,
    mesh=_mesh,
    out_type=(jax.ShapeDtypeStruct((NC, N, D), jnp.float32),
              jax.ShapeDtypeStruct((NW, N), jnp.float32)),
    scratch_types=[
        pltpu.VMEM((N,), jnp.float32),
        pltpu.VMEM((N,), jnp.float32),
        pltpu.VMEM((EW,), jnp.int32),
        pltpu.VMEM((NB, K), jnp.int32),
        pltpu.VMEM((EW,), jnp.float32),
        pltpu.VMEM((N,), jnp.float32),
        pltpu.VMEM((K,), jnp.float32),
        pltpu.VMEM((K, D), jnp.float32),
        pltpu.VMEM_SHARED((N, D), jnp.float32),
        pltpu.SemaphoreType.DMA,
    ],
)
def _edge_agg(p_hbm, q_hbm, src_hbm, dst3_hbm, lab_hbm, zr_hbm, v_hbm,
              agg_hbm, den_hbm,
              p_v, q_v, s_v, d2_v, l_v, den_v, wblk_v, rows_v,
              agg_s, sem):
    cid = lax.axis_index("c")
    sid = lax.axis_index("s")
    wid = sid * NC + cid
    base = wid * EW

    pltpu.sync_copy(p_hbm, p_v)
    pltpu.sync_copy(q_hbm, q_v)
    pltpu.sync_copy(src_hbm.at[pl.ds(base, EW)], s_v)
    pltpu.sync_copy(dst3_hbm.at[wid], d2_v)
    pltpu.sync_copy(lab_hbm.at[pl.ds(base, EW)], l_v)
    pltpu.sync_copy(zr_hbm, agg_s.at[pl.ds(sid * RPT, RPT)])

    def zero_body(i, _):
        den_v[pl.ds(i * L, L)] = jnp.zeros((L,), jnp.float32)
        return 0
    lax.fori_loop(0, N // L, zero_body, 0)

    plsc.subcore_barrier()

    def body(b, _):
        gather = pltpu.make_async_copy(
            v_hbm.at[s_v.at[pl.ds(b * K, K)]], rows_v, sem)
        gather.start()
        idst = d2_v[b]
        isrc = s_v[pl.ds(b * K, K)]
        e = plsc.load_gather(p_v, [idst]) + plsc.load_gather(q_v, [isrc])
        e = jnp.where(e > 0, e, 0.2 * e)
        m = l_v[pl.ds(b * K, K)] > 0.5
        w = jnp.where(m, jnp.exp(e), 0.0)
        plsc.addupdate_scatter(den_v, [idst], w)
        wblk_v[...] = w
        gather.wait()
        for j in range(K):
            wj = wblk_v[j]
            for c in range(D // L):
                sl = pl.ds(c * L, L)
                rows_v[j, sl] = rows_v[j, sl] * wj
        pltpu.sync_copy(rows_v, agg_s.at[d2_v.at[b]], add=True)
        return 0
    lax.fori_loop(0, NB, body, 0)

    plsc.subcore_barrier()
    pltpu.sync_copy(agg_s.at[pl.ds(sid * RPT, RPT)],
                    agg_hbm.at[cid, pl.ds(sid * RPT, RPT)])
    pltpu.sync_copy(den_v, den_hbm.at[wid])


def _fin_body(x_ref, agg_ref, den_ref, o_ref):
    den = jnp.sum(den_ref[...], axis=0)[:, None]
    agg = agg_ref[0] + agg_ref[1]
    safe = jnp.where(den > 0, den, 1.0)
    o_ref[...] = x_ref[...] + jnp.where(den > 0, agg / safe, 0.0)


_fin = pl.pallas_call(
    _fin_body,
    out_shape=jax.ShapeDtypeStruct((N, D), jnp.float32),
)


def kernel(event_embeddings, event_pairs, coreference_labels,
           W_V_w, W_V_b, W_R_w, a_w, a_b):
    src = event_pairs[:, 0]
    dst3 = event_pairs[:, 1].reshape(NW, NB, K)
    lab = coreference_labels[:, 0]
    ab = jnp.reshape(a_b, (1,))
    V, p, q = _dense(event_embeddings, W_V_w, W_R_w, W_V_b, a_w, ab)
    zr = jnp.zeros((RPT, D), jnp.float32)
    agg, den = _edge_agg(p, q, src, dst3, lab, zr, V)
    return _fin(event_embeddings, agg, den)


# speedup vs baseline: 9.9629x; 9.9629x over previous
import functools

import jax
import jax.numpy as jnp
from jax import lax
from jax.experimental import pallas as pl
from jax.experimental.pallas import tpu as pltpu
from jax.experimental.pallas import tpu_sc as plsc

N = 10000
D = 128
E = 320000
NC = 2
NS = 16
NW = NC * NS
EW = E // NW
L = 16
K = 16
SEGE = 400
NSEG = EW // SEGE
NBS = SEGE // K
NPAD = 10112
RPT = NPAD // NS


def _dense_body(x_ref, wv_ref, wr_ref, bv_ref, aw_ref, ab_ref,
                v_ref, p_ref, q_ref):
    x = x_ref[...]
    v_ref[...] = lax.dot_general(
        x, wv_ref[...], (((1,), (1,)), ((), ())),
        precision=lax.Precision.HIGHEST,
        preferred_element_type=jnp.float32) + bv_ref[...][None, :]
    a1 = aw_ref[0:D][None, :]
    a2 = aw_ref[D:2 * D][None, :]
    w2 = lax.dot_general(a2, wr_ref[...], (((1,), (0,)), ((), ())),
                         precision=lax.Precision.HIGHEST,
                         preferred_element_type=jnp.float32)
    p_ref[...] = lax.dot_general(a1, x, (((1,), (1,)), ((), ())),
                                 precision=lax.Precision.HIGHEST,
                                 preferred_element_type=jnp.float32) + ab_ref[0]
    q_ref[...] = lax.dot_general(w2, x, (((1,), (1,)), ((), ())),
                                 precision=lax.Precision.HIGHEST,
                                 preferred_element_type=jnp.float32)


_dense = pl.pallas_call(
    _dense_body,
    out_shape=(jax.ShapeDtypeStruct((N, D), jnp.float32),
               jax.ShapeDtypeStruct((1, N), jnp.float32),
               jax.ShapeDtypeStruct((1, N), jnp.float32)),
    in_specs=[pl.BlockSpec(memory_space=pltpu.VMEM)] * 5
             + [pl.BlockSpec(memory_space=pltpu.SMEM)],
)


_mesh = plsc.VectorSubcoreMesh(core_axis_name="c", subcore_axis_name="s")


@functools.partial(
    pl.kernel,
    mesh=_mesh,
    compiler_params=pltpu.CompilerParams(needs_layout_passes=False),
    out_type=(jax.ShapeDtypeStruct((NC, NPAD, D), jnp.float32),
              jax.ShapeDtypeStruct((NW, 1, N), jnp.float32)),
    scratch_types=[
        pltpu.VMEM((1, N), jnp.float32),
        pltpu.VMEM((1, N), jnp.float32),
        pltpu.VMEM((SEGE,), jnp.int32),
        pltpu.VMEM((NBS, K), jnp.int32),
        pltpu.VMEM((SEGE,), jnp.float32),
        pltpu.VMEM((1, N), jnp.float32),
        pltpu.VMEM((K, D), jnp.float32),
        pltpu.VMEM_SHARED((NPAD, D), jnp.float32),
        pltpu.SemaphoreType.DMA,
    ],
)
def _edge_agg(p_hbm, q_hbm, src_hbm, dst4_hbm, lab_hbm, zr_hbm, v_hbm,
              agg_hbm, den_hbm,
              p_v, q_v, s_v, d2_v, l_v, den_v, rows_v,
              agg_s, sem):
    cid = lax.axis_index("c")
    sid = lax.axis_index("s")
    wid = sid * NC + cid
    base = wid * EW

    pltpu.sync_copy(p_hbm, p_v)
    pltpu.sync_copy(q_hbm, q_v)
    pltpu.sync_copy(zr_hbm, agg_s.at[pl.ds(sid * RPT, RPT)])

    def zero_body(i, _):
        den_v[0, pl.ds(i * L, L)] = jnp.zeros((L,), jnp.float32)
        return 0
    lax.fori_loop(0, N // L, zero_body, 0)

    plsc.subcore_barrier()

    def seg_body(g, _):
        seg_base = base + g * SEGE
        pltpu.sync_copy(src_hbm.at[pl.ds(seg_base, SEGE)], s_v)
        pltpu.sync_copy(dst4_hbm.at[wid, g], d2_v)
        pltpu.sync_copy(lab_hbm.at[pl.ds(seg_base, SEGE)], l_v)

        def body(b, _):
            gather = pltpu.make_async_copy(
                v_hbm.at[s_v.at[pl.ds(b * K, K)]], rows_v, sem)
            gather.start()
            idst = d2_v[b]
            isrc = s_v[pl.ds(b * K, K)]
            zz = jnp.zeros((L,), jnp.int32)
            e = (plsc.load_gather(p_v, [zz, idst])
                 + plsc.load_gather(q_v, [zz, isrc]))
            e = jnp.where(e > 0, e, 0.2 * e)
            m = l_v[pl.ds(b * K, K)] > 0.5
            w = jnp.where(m, jnp.exp(e), 0.0)
            plsc.addupdate_scatter(den_v, [zz, idst], w)
            gather.wait()
            for j in range(K):
                wj = w[j]
                for c in range(D // L):
                    sl = pl.ds(c * L, L)
                    rows_v[j, sl] = rows_v[j, sl] * wj
            pltpu.sync_copy(rows_v, agg_s.at[d2_v.at[b]], add=True)
            return 0
        lax.fori_loop(0, NBS, body, 0)
        return 0
    lax.fori_loop(0, NSEG, seg_body, 0)

    plsc.subcore_barrier()
    pltpu.sync_copy(agg_s.at[pl.ds(sid * RPT, RPT)],
                    agg_hbm.at[cid, pl.ds(sid * RPT, RPT)])
    pltpu.sync_copy(den_v, den_hbm.at[wid])


def _fin_body(x_ref, agg_ref, den_ref, o_ref):
    den = jnp.sum(den_ref[...], axis=(0, 1))[:, None]
    agg = (agg_ref[0] + agg_ref[1])[:N]
    safe = jnp.where(den > 0, den, 1.0)
    o_ref[...] = x_ref[...] + jnp.where(den > 0, agg / safe, 0.0)


_fin = pl.pallas_call(
    _fin_body,
    out_shape=jax.ShapeDtypeStruct((N, D), jnp.float32),
)


def kernel(event_embeddings, event_pairs, coreference_labels,
           W_V_w, W_V_b, W_R_w, a_w, a_b):
    src = event_pairs[:, 0]
    dst4 = event_pairs[:, 1].reshape(NW, NSEG, NBS, K)
    lab = coreference_labels[:, 0]
    ab = jnp.reshape(a_b, (1,))
    V, p, q = _dense(event_embeddings, W_V_w, W_R_w, W_V_b, a_w, ab)
    zr = jnp.zeros((RPT, D), jnp.float32)
    agg, den = _edge_agg(p, q, src, dst4, lab, zr, V)
    return _fin(event_embeddings, agg, den)


# speedup vs baseline: 22.5018x; 2.2586x over previous
import functools

import jax
import jax.numpy as jnp
from jax import lax
from jax.experimental import pallas as pl
from jax.experimental.pallas import tpu as pltpu
from jax.experimental.pallas import tpu_sc as plsc

N = 10000
D = 128
E = 320000
NC = 2
NS = 16
NW = NC * NS
EW = E // NW
L = 16
K = 16
SEGE = 2000
NSEG = EW // SEGE
NBS = SEGE // K
R = 5
G = 2
NPAD = 10112
RPT = NPAD // NS


def _dense_body(x_ref, wv_ref, wr_ref, bv_ref, aw_ref, ab_ref,
                v_ref, p_ref, q_ref):
    x = x_ref[...]
    v_ref[...] = lax.dot_general(
        x, wv_ref[...], (((1,), (1,)), ((), ())),
        precision=lax.Precision.HIGHEST,
        preferred_element_type=jnp.float32) + bv_ref[...][None, :]
    a1 = aw_ref[0:D][None, :]
    a2 = aw_ref[D:2 * D][None, :]
    w2 = lax.dot_general(a2, wr_ref[...], (((1,), (0,)), ((), ())),
                         precision=lax.Precision.HIGHEST,
                         preferred_element_type=jnp.float32)
    p_ref[...] = lax.dot_general(a1, x, (((1,), (1,)), ((), ())),
                                 precision=lax.Precision.HIGHEST,
                                 preferred_element_type=jnp.float32) + ab_ref[0]
    q_ref[...] = lax.dot_general(w2, x, (((1,), (1,)), ((), ())),
                                 precision=lax.Precision.HIGHEST,
                                 preferred_element_type=jnp.float32)


_dense = pl.pallas_call(
    _dense_body,
    out_shape=(jax.ShapeDtypeStruct((N, D), jnp.float32),
               jax.ShapeDtypeStruct((1, N), jnp.float32),
               jax.ShapeDtypeStruct((1, N), jnp.float32)),
    in_specs=[pl.BlockSpec(memory_space=pltpu.VMEM)] * 5
             + [pl.BlockSpec(memory_space=pltpu.SMEM)],
)


_mesh = plsc.VectorSubcoreMesh(core_axis_name="c", subcore_axis_name="s")


@functools.partial(
    pl.kernel,
    mesh=_mesh,
    compiler_params=pltpu.CompilerParams(needs_layout_passes=False),
    out_type=(jax.ShapeDtypeStruct((E,), jnp.float32),
              jax.ShapeDtypeStruct((NW, 1, N), jnp.float32)),
    scratch_types=[
        pltpu.VMEM((1, N), jnp.float32),
        pltpu.VMEM((1, N), jnp.float32),
        pltpu.VMEM((SEGE,), jnp.int32),
        pltpu.VMEM((SEGE,), jnp.int32),
        pltpu.VMEM((SEGE,), jnp.float32),
        pltpu.VMEM((SEGE,), jnp.float32),
        pltpu.VMEM((1, N), jnp.float32),
    ],
)
def _edge_w(p_hbm, q_hbm, src_hbm, dst_hbm, lab_hbm,
            w_hbm, den_hbm,
            p_v, q_v, s_v, d_v, l_v, w_v, den_v):
    cid = lax.axis_index("c")
    sid = lax.axis_index("s")
    wid = sid * NC + cid
    base = wid * EW

    pltpu.sync_copy(p_hbm, p_v)
    pltpu.sync_copy(q_hbm, q_v)

    def zero_body(i, _):
        den_v[0, pl.ds(i * L, L)] = jnp.zeros((L,), jnp.float32)
        return 0
    lax.fori_loop(0, N // L, zero_body, 0)

    def seg_body(g, _):
        seg_base = base + g * SEGE
        pltpu.sync_copy(src_hbm.at[pl.ds(seg_base, SEGE)], s_v)
        pltpu.sync_copy(dst_hbm.at[pl.ds(seg_base, SEGE)], d_v)
        pltpu.sync_copy(lab_hbm.at[pl.ds(seg_base, SEGE)], l_v)

        def body(b, _):
            sl = pl.ds(b * K, K)
            idst = d_v[sl]
            isrc = s_v[sl]
            zz = jnp.zeros((L,), jnp.int32)
            e = (plsc.load_gather(p_v, [zz, idst])
                 + plsc.load_gather(q_v, [zz, isrc]))
            e = jnp.where(e > 0, e, 0.2 * e)
            m = l_v[sl] > 0.5
            w = jnp.where(m, jnp.exp(e), 0.0)
            w_v[sl] = w
            plsc.addupdate_scatter(den_v, [zz, idst], w)
            return 0
        lax.fori_loop(0, NBS, body, 0)
        pltpu.sync_copy(w_v, w_hbm.at[pl.ds(seg_base, SEGE)])
        return 0
    lax.fori_loop(0, NSEG, seg_body, 0)

    pltpu.sync_copy(den_v, den_hbm.at[wid])


@functools.partial(
    pl.kernel,
    mesh=_mesh,
    compiler_params=pltpu.CompilerParams(needs_layout_passes=False),
    out_type=jax.ShapeDtypeStruct((NC, NPAD, D), jnp.float32),
    scratch_types=[
        pltpu.VMEM((SEGE,), jnp.int32),
        pltpu.VMEM((NBS, K), jnp.int32),
        pltpu.VMEM((SEGE,), jnp.float32),
        pltpu.VMEM((R, K, D), jnp.float32),
        pltpu.VMEM_SHARED((NPAD, D), jnp.float32),
        [pltpu.SemaphoreType.DMA] * R,
        [pltpu.SemaphoreType.DMA] * R,
    ],
)
def _edge_agg(src_hbm, dst4_hbm, w_hbm, zr_hbm, v_hbm,
              agg_hbm,
              s_v, d2_v, w_v, rows_v, agg_s, gsem, ssem):
    cid = lax.axis_index("c")
    sid = lax.axis_index("s")
    wid = sid * NC + cid
    base = wid * EW

    pltpu.sync_copy(zr_hbm, agg_s.at[pl.ds(sid * RPT, RPT)])
    plsc.subcore_barrier()

    def _gather_desc(b, slot):
        return pltpu.make_async_copy(
            v_hbm.at[s_v.at[pl.ds(b * K, K)]], rows_v.at[slot], gsem[slot])

    def _scatter_desc(b, slot):
        return pltpu.make_async_copy(
            rows_v.at[slot], agg_s.at[d2_v.at[b]], ssem[slot])

    def seg_body(g, _):
        seg_base = base + g * SEGE
        pltpu.sync_copy(src_hbm.at[pl.ds(seg_base, SEGE)], s_v)
        pltpu.sync_copy(dst4_hbm.at[wid, g], d2_v)
        pltpu.sync_copy(w_hbm.at[pl.ds(seg_base, SEGE)], w_v)

        for b0 in range(G):
            _gather_desc(b0, b0).start()

        def body(grp, _):
            for slot in range(R):
                b = grp * R + slot
                @pl.when(b >= R - G)
                def _():
                    _scatter_desc(b - (R - G), (slot + G) % R).wait()

                @pl.when(b + G < NBS)
                def _():
                    _gather_desc(b + G, (slot + G) % R).start()

                w = w_v[pl.ds(b * K, K)]
                _gather_desc(b, slot).wait()
                for j in range(K):
                    wj = w[j]
                    for c in range(D // L):
                        cc = pl.ds(c * L, L)
                        rows_v[slot, j, cc] = rows_v[slot, j, cc] * wj
                _scatter_desc(b, slot).start(add=True)
            return 0
        lax.fori_loop(0, NBS // R, body, 0)

        for tail in range(R - G):
            b = NBS - (R - G) + tail
            _scatter_desc(b, b % R).wait()
        return 0
    lax.fori_loop(0, NSEG, seg_body, 0)

    plsc.subcore_barrier()
    pltpu.sync_copy(agg_s.at[pl.ds(sid * RPT, RPT)],
                    agg_hbm.at[cid, pl.ds(sid * RPT, RPT)])


def _fin_body(x_ref, agg_ref, den_ref, o_ref):
    den = jnp.sum(den_ref[...], axis=(0, 1))[:, None]
    agg = (agg_ref[0] + agg_ref[1])[:N]
    safe = jnp.where(den > 0, den, 1.0)
    o_ref[...] = x_ref[...] + jnp.where(den > 0, agg / safe, 0.0)


_fin = pl.pallas_call(
    _fin_body,
    out_shape=jax.ShapeDtypeStruct((N, D), jnp.float32),
)


def kernel(event_embeddings, event_pairs, coreference_labels,
           W_V_w, W_V_b, W_R_w, a_w, a_b):
    src = event_pairs[:, 0]
    dst = event_pairs[:, 1]
    dst4 = dst.reshape(NW, NSEG, NBS, K)
    lab = coreference_labels[:, 0]
    ab = jnp.reshape(a_b, (1,))
    V, p, q = _dense(event_embeddings, W_V_w, W_R_w, W_V_b, a_w, ab)
    w, den = _edge_w(p, q, src, dst, lab)
    zr = jnp.zeros((RPT, D), jnp.float32)
    agg = _edge_agg(src, dst4, w, zr, V)
    return _fin(event_embeddings, agg, den)


# speedup vs baseline: 25.4655x; 1.1317x over previous
import functools

import jax
import jax.numpy as jnp
from jax import lax
from jax.experimental import pallas as pl
from jax.experimental.pallas import tpu as pltpu
from jax.experimental.pallas import tpu_sc as plsc

N = 10000
D = 128
E = 320000
NC = 2
NS = 16
NW = NC * NS
EW = E // NW
L = 16
K = 16
SEGE = 2000
NSEG = EW // SEGE
NBS = SEGE // K
R = 5
G = 3
NPAD = 10112
RPT = NPAD // NS


def _dense_body(x_ref, wv_ref, wr_ref, bv_ref, aw_ref, ab_ref,
                v_ref, p_ref, q_ref):
    x = x_ref[...]
    v_ref[...] = lax.dot_general(
        x, wv_ref[...], (((1,), (1,)), ((), ())),
        precision=lax.Precision.HIGHEST,
        preferred_element_type=jnp.float32) + bv_ref[...][None, :]
    a1 = aw_ref[0:D][None, :]
    a2 = aw_ref[D:2 * D][None, :]
    w2 = lax.dot_general(a2, wr_ref[...], (((1,), (0,)), ((), ())),
                         precision=lax.Precision.HIGHEST,
                         preferred_element_type=jnp.float32)
    p_ref[...] = lax.dot_general(a1, x, (((1,), (1,)), ((), ())),
                                 precision=lax.Precision.HIGHEST,
                                 preferred_element_type=jnp.float32) + ab_ref[0]
    q_ref[...] = lax.dot_general(w2, x, (((1,), (1,)), ((), ())),
                                 precision=lax.Precision.HIGHEST,
                                 preferred_element_type=jnp.float32)


_dense = pl.pallas_call(
    _dense_body,
    out_shape=(jax.ShapeDtypeStruct((N, D), jnp.float32),
               jax.ShapeDtypeStruct((1, N), jnp.float32),
               jax.ShapeDtypeStruct((1, N), jnp.float32)),
    in_specs=[pl.BlockSpec(memory_space=pltpu.VMEM)] * 5
             + [pl.BlockSpec(memory_space=pltpu.SMEM)],
)


_mesh = plsc.VectorSubcoreMesh(core_axis_name="c", subcore_axis_name="s")


@functools.partial(
    pl.kernel,
    mesh=_mesh,
    compiler_params=pltpu.CompilerParams(needs_layout_passes=False),
    out_type=(jax.ShapeDtypeStruct((E,), jnp.float32),
              jax.ShapeDtypeStruct((NW, 1, N), jnp.float32)),
    scratch_types=[
        pltpu.VMEM((1, N), jnp.float32),
        pltpu.VMEM((1, N), jnp.float32),
        pltpu.VMEM((SEGE,), jnp.int32),
        pltpu.VMEM((SEGE,), jnp.int32),
        pltpu.VMEM((SEGE,), jnp.float32),
        pltpu.VMEM((SEGE,), jnp.float32),
        pltpu.VMEM((1, N), jnp.float32),
    ],
)
def _edge_w(p_hbm, q_hbm, src_hbm, dst_hbm, lab_hbm,
            w_hbm, den_hbm,
            p_v, q_v, s_v, d_v, l_v, w_v, den_v):
    cid = lax.axis_index("c")
    sid = lax.axis_index("s")
    wid = sid * NC + cid
    base = wid * EW

    pltpu.sync_copy(p_hbm, p_v)
    pltpu.sync_copy(q_hbm, q_v)

    def zero_body(i, _):
        den_v[0, pl.ds(i * L, L)] = jnp.zeros((L,), jnp.float32)
        return 0
    lax.fori_loop(0, N // L, zero_body, 0)

    def seg_body(g, _):
        seg_base = base + g * SEGE
        pltpu.sync_copy(src_hbm.at[pl.ds(seg_base, SEGE)], s_v)
        pltpu.sync_copy(dst_hbm.at[pl.ds(seg_base, SEGE)], d_v)
        pltpu.sync_copy(lab_hbm.at[pl.ds(seg_base, SEGE)], l_v)

        def body(b, _):
            sl = pl.ds(b * K, K)
            idst = d_v[sl]
            isrc = s_v[sl]
            zz = jnp.zeros((L,), jnp.int32)
            e = (plsc.load_gather(p_v, [zz, idst])
                 + plsc.load_gather(q_v, [zz, isrc]))
            e = jnp.where(e > 0, e, 0.2 * e)
            m = l_v[sl] > 0.5
            w = jnp.where(m, jnp.exp(e), 0.0)
            w_v[sl] = w
            plsc.addupdate_scatter(den_v, [zz, idst], w)
            return 0
        lax.fori_loop(0, NBS, body, 0)
        pltpu.sync_copy(w_v, w_hbm.at[pl.ds(seg_base, SEGE)])
        return 0
    lax.fori_loop(0, NSEG, seg_body, 0)

    pltpu.sync_copy(den_v, den_hbm.at[wid])


@functools.partial(
    pl.kernel,
    mesh=_mesh,
    compiler_params=pltpu.CompilerParams(needs_layout_passes=False),
    out_type=jax.ShapeDtypeStruct((NC, NPAD, D), jnp.float32),
    scratch_types=[
        pltpu.VMEM((SEGE,), jnp.int32),
        pltpu.VMEM((NBS, K), jnp.int32),
        pltpu.VMEM((SEGE,), jnp.float32),
        pltpu.VMEM((R, K, D), jnp.float32),
        pltpu.VMEM_SHARED((NPAD, D), jnp.float32),
        [pltpu.SemaphoreType.DMA] * R,
        [pltpu.SemaphoreType.DMA] * R,
    ],
)
def _edge_agg(src_hbm, dst4_hbm, w_hbm, zr_hbm, v_hbm,
              agg_hbm,
              s_v, d2_v, w_v, rows_v, agg_s, gsem, ssem):
    cid = lax.axis_index("c")
    sid = lax.axis_index("s")
    wid = sid * NC + cid
    base = wid * EW

    pltpu.sync_copy(zr_hbm, agg_s.at[pl.ds(sid * RPT, RPT)])
    plsc.subcore_barrier()

    def _gather_desc(b, slot):
        return pltpu.make_async_copy(
            v_hbm.at[s_v.at[pl.ds(b * K, K)]], rows_v.at[slot], gsem[slot])

    def _scatter_desc(b, slot):
        return pltpu.make_async_copy(
            rows_v.at[slot], agg_s.at[d2_v.at[b]], ssem[slot])

    def seg_body(g, _):
        seg_base = base + g * SEGE
        pltpu.sync_copy(src_hbm.at[pl.ds(seg_base, SEGE)], s_v)
        pltpu.sync_copy(dst4_hbm.at[wid, g], d2_v)
        pltpu.sync_copy(w_hbm.at[pl.ds(seg_base, SEGE)], w_v)

        for b0 in range(G):
            _gather_desc(b0, b0).start()

        def body(grp, _):
            for slot in range(R):
                b = grp * R + slot
                @pl.when(b >= R - G)
                def _():
                    _scatter_desc(b - (R - G), (slot + G) % R).wait()

                @pl.when(b + G < NBS)
                def _():
                    _gather_desc(b + G, (slot + G) % R).start()

                w = w_v[pl.ds(b * K, K)]
                _gather_desc(b, slot).wait()
                for j in range(K):
                    wj = w[j]
                    for c in range(D // L):
                        cc = pl.ds(c * L, L)
                        rows_v[slot, j, cc] = rows_v[slot, j, cc] * wj
                _scatter_desc(b, slot).start(add=True)
            return 0
        lax.fori_loop(0, NBS // R, body, 0)

        for tail in range(R - G):
            b = NBS - (R - G) + tail
            _scatter_desc(b, b % R).wait()
        return 0
    lax.fori_loop(0, NSEG, seg_body, 0)

    plsc.subcore_barrier()
    pltpu.sync_copy(agg_s.at[pl.ds(sid * RPT, RPT)],
                    agg_hbm.at[cid, pl.ds(sid * RPT, RPT)])


def _fin_body(x_ref, agg_ref, den_ref, o_ref):
    den = jnp.sum(den_ref[...], axis=(0, 1))[:, None]
    agg = (agg_ref[0] + agg_ref[1])[:N]
    safe = jnp.where(den > 0, den, 1.0)
    o_ref[...] = x_ref[...] + jnp.where(den > 0, agg / safe, 0.0)


_fin = pl.pallas_call(
    _fin_body,
    out_shape=jax.ShapeDtypeStruct((N, D), jnp.float32),
)


def kernel(event_embeddings, event_pairs, coreference_labels,
           W_V_w, W_V_b, W_R_w, a_w, a_b):
    src = event_pairs[:, 0]
    dst = event_pairs[:, 1]
    dst4 = dst.reshape(NW, NSEG, NBS, K)
    lab = coreference_labels[:, 0]
    ab = jnp.reshape(a_b, (1,))
    V, p, q = _dense(event_embeddings, W_V_w, W_R_w, W_V_b, a_w, ab)
    w, den = _edge_w(p, q, src, dst, lab)
    zr = jnp.zeros((RPT, D), jnp.float32)
    agg = _edge_agg(src, dst4, w, zr, V)
    return _fin(event_embeddings, agg, den)
